# baseline (speedup 1.0000x reference)
import functools
import math

import jax
import jax.numpy as jnp
from jax import lax
from jax.experimental import pallas as pl
from jax.experimental.pallas import tpu as pltpu
from jax.experimental.pallas import tpu_sc as plsc

_INV_SQRT2 = 1.0 / math.sqrt(2.0)
_PREC = lax.Precision.HIGHEST


def _edge_body(basisT_ref, m_ref, w_ref, x_ref):
    emb = lax.dot_general(
        basisT_ref[...], w_ref[...], (((0,), (0,)), ((), ())),
        preferred_element_type=jnp.float32, precision=_PREC)
    x_ref[...] = m_ref[...] * emb


@functools.lru_cache(maxsize=None)
def _edge_fuse(nE_chunk, base_blk, dR, dE, block_e):
    return pl.pallas_call(
        _edge_body,
        grid=(nE_chunk // block_e,),
        in_specs=[
            pl.BlockSpec((dR, block_e), lambda i: (0, i + base_blk)),
            pl.BlockSpec((block_e, dE), lambda i: (i + base_blk, 0)),
            pl.BlockSpec((dR, dE), lambda i: (0, 0)),
        ],
        out_specs=pl.BlockSpec((block_e, dE), lambda i: (i, 0)),
        out_shape=jax.ShapeDtypeStruct((nE_chunk, dE), jnp.float32),
    )


@functools.lru_cache(maxsize=None)
def _sc_scatter(nE_chunk, idx_off, nA, D):
    NC, NS = 2, 16
    NW = NC * NS
    epw = nE_chunk // NW
    block = 192
    nblk = epw // block
    tail = epw % block
    assert tail % 8 == 0 and tail <= 128
    rows_lo = (nA // NS) // 8 * 8
    rows_hi = nA - rows_lo * (NS - 1)

    mesh = plsc.VectorSubcoreMesh(core_axis_name="c", subcore_axis_name="s")

    scratch = [
        pltpu.VMEM((block, D), jnp.float32),
        pltpu.VMEM((block, D), jnp.float32),
        pltpu.VMEM((128,), jnp.int32),
        pltpu.VMEM((64,), jnp.int32),
        pltpu.VMEM((128,), jnp.int32),
        pltpu.VMEM((64,), jnp.int32),
        pltpu.SemaphoreType.DMA,
        pltpu.SemaphoreType.DMA,
        pltpu.VMEM_SHARED((nA, D), jnp.float32),
    ]
    if tail:
        scratch.insert(6, pltpu.VMEM((tail,), jnp.int32))

    @functools.partial(
        pl.kernel,
        mesh=mesh,
        out_type=jax.ShapeDtypeStruct((NC, nA, D), jnp.float32),
        scratch_types=scratch,
    )
    def sc_scatter(x_hbm, idx_hbm, out_hbm, *refs):
        if tail:
            xb0, xb1, ia0, ib0, ia1, ib1, it, sem0, sem1, acc = refs
        else:
            xb0, xb1, ia0, ib0, ia1, ib1, sem0, sem1, acc = refs
        xbufs = (xb0, xb1)
        ibufs = ((ia0, ib0), (ia1, ib1))
        sems = (sem0, sem1)
        c = lax.axis_index("c")
        s = lax.axis_index("s")
        wid = c * NS + s

        def _zrow(r, carry):
            for j in range(D // 16):
                xb0[r, pl.ds(j * 16, 16)] = jnp.zeros((16,), jnp.float32)
            return carry
        lax.fori_loop(0, block, _zrow, 0)

        r0 = s * rows_lo

        def _zero_rows(n):
            for k in range(n // block):
                pltpu.sync_copy(xb0, acc.at[pl.ds(r0 + k * block, block)])
            rem = n % block
            if rem:
                pltpu.sync_copy(xb0.at[pl.ds(0, rem)],
                                acc.at[pl.ds(r0 + (n // block) * block, rem)])

        @pl.when(s < NS - 1)
        def _():
            _zero_rows(rows_lo)

        @pl.when(s == NS - 1)
        def _():
            _zero_rows(rows_hi)

        plsc.subcore_barrier()

        base_w = wid * epw

        def _start(b):
            buf = b % 2
            ebase = base_w + b * block
            return (
                pltpu.async_copy(x_hbm.at[pl.ds(ebase, block)],
                                 xbufs[buf], sems[buf]),
                pltpu.async_copy(idx_hbm.at[pl.ds(idx_off + ebase, 128)],
                                 ibufs[buf][0], sems[buf]),
                pltpu.async_copy(idx_hbm.at[pl.ds(idx_off + ebase + 128, 64)],
                                 ibufs[buf][1], sems[buf]),
            )

        def _drain(handles):
            for hd in handles:
                hd.wait()

        def _scatter(b):
            buf = b % 2
            pltpu.sync_copy(xbufs[buf].at[pl.ds(0, 128)],
                            acc.at[ibufs[buf][0]], add=True)
            pltpu.sync_copy(xbufs[buf].at[pl.ds(128, 64)],
                            acc.at[ibufs[buf][1]], add=True)

        if nblk:
            pending = _start(0)
            for b in range(nblk):
                nxt = _start(b + 1) if b + 1 < nblk else None
                _drain(pending)
                _scatter(b)
                pending = nxt
        if tail:
            ebase = base_w + nblk * block
            pltpu.sync_copy(x_hbm.at[pl.ds(ebase, tail)],
                            xb0.at[pl.ds(0, tail)])
            pltpu.sync_copy(idx_hbm.at[pl.ds(idx_off + ebase, tail)], it)
            pltpu.sync_copy(xb0.at[pl.ds(0, tail)], acc.at[it], add=True)

        plsc.subcore_barrier()

        def _write_rows(n):
            nfull = n // block
            hs = [None, None]
            for k in range(nfull):
                off = r0 + k * block
                buf = k % 2
                if hs[buf] is not None:
                    hs[buf].wait()
                pltpu.sync_copy(acc.at[pl.ds(off, block)], xbufs[buf])
                hs[buf] = pltpu.async_copy(
                    xbufs[buf], out_hbm.at[c, pl.ds(off, block)], sems[buf])
            rem = n % block
            if rem:
                off = r0 + nfull * block
                buf = nfull % 2
                if hs[buf] is not None:
                    hs[buf].wait()
                pltpu.sync_copy(acc.at[pl.ds(off, rem)],
                                xbufs[buf].at[pl.ds(0, rem)])
                hs[buf] = pltpu.async_copy(
                    xbufs[buf].at[pl.ds(0, rem)],
                    out_hbm.at[c, pl.ds(off, rem)], sems[buf])
            for hd in hs:
                if hd is not None:
                    hd.wait()

        @pl.when(s < NS - 1)
        def _():
            _write_rows(rows_lo)

        @pl.when(s == NS - 1)
        def _():
            _write_rows(rows_hi)

    return sc_scatter


def _dot3(x, w):
    xh = x.astype(jnp.bfloat16)
    xl = (x - xh.astype(jnp.float32)).astype(jnp.bfloat16)
    wh = w.astype(jnp.bfloat16)
    wl = (w - wh.astype(jnp.float32)).astype(jnp.bfloat16)

    def d(a, b):
        return jax.lax.dot_general(
            a, b, (((1,), (0,)), ((), ())),
            preferred_element_type=jnp.float32)

    return d(xh, wh) + (d(xh, wl) + d(xl, wh))


def _res_layer(x, W1, W2):
    y = _dot3(x, W1)
    y = y * jax.nn.sigmoid(y)
    y = _dot3(y, W2)
    y = y * jax.nn.sigmoid(y)
    return (x + y) * _INV_SQRT2


def _mlp_body_fixed(n_parts):
    def body(*refs):
        part_refs = refs[:n_parts]
        h_ref, wpre_ref, wpost_ref, scale_ref, out_ref = refs[n_parts:]
        x = part_refs[0][0] + part_refs[0][1]
        for p in part_refs[1:]:
            x = x + p[0] + p[1]
        x = x * scale_ref[0, 0]
        for i in range(wpre_ref.shape[0]):
            x = _res_layer(x, wpre_ref[i, 0], wpre_ref[i, 1])
        x = (x + h_ref[...]) * _INV_SQRT2
        for i in range(wpost_ref.shape[0]):
            x = _res_layer(x, wpost_ref[i, 0], wpost_ref[i, 1])
        out_ref[...] = x
    return body


@functools.lru_cache(maxsize=None)
def _mlp(n_parts, nA, D, nPre, nPost, block_a):
    part_specs = [pl.BlockSpec((2, block_a, D), lambda i: (0, i, 0))
                  for _ in range(n_parts)]
    return pl.pallas_call(
        _mlp_body_fixed(n_parts),
        grid=(nA // block_a,),
        in_specs=part_specs + [
            pl.BlockSpec((block_a, D), lambda i: (i, 0)),
            pl.BlockSpec((nPre, 2, D, D), lambda i: (0, 0, 0, 0)),
            pl.BlockSpec((nPost, 2, D, D), lambda i: (0, 0, 0, 0)),
            pl.BlockSpec((1, 1), lambda i: (0, 0), memory_space=pltpu.SMEM),
        ],
        out_specs=pl.BlockSpec((block_a, D), lambda i: (i, 0)),
        out_shape=jax.ShapeDtypeStruct((nA, D), jnp.float32),
    )


_CHUNKS = (102400, 89600, 76800, 51200)
_BLOCK_E = 12800


def kernel(h, m, basis_rad, idx_atom, W_rbf, W_pre, W_post, scale):
    nA, D = h.shape
    nE, dR = basis_rad.shape
    idx32 = idx_atom.astype(jnp.int32)
    basisT = basis_rad.T
    partials = []
    base = 0
    for chunk in _CHUNKS:
        x_k = _edge_fuse(chunk, base // _BLOCK_E, dR, D, _BLOCK_E)(
            basisT, m, W_rbf)
        partials.append(_sc_scatter(chunk, base, nA, D)(x_k, idx32))
        base += chunk
    out = _mlp(len(partials), nA, D, W_pre.shape[0], W_post.shape[0], 5000)(
        *partials, h, W_pre, W_post, scale.reshape(1, 1))
    return out


# speedup vs baseline: 1.0204x; 1.0204x over previous
import functools
import math

import jax
import jax.numpy as jnp
from jax import lax
from jax.experimental import pallas as pl
from jax.experimental.pallas import tpu as pltpu
from jax.experimental.pallas import tpu_sc as plsc

_INV_SQRT2 = 1.0 / math.sqrt(2.0)
_PREC = lax.Precision.HIGHEST


def _edge_body(basisT_ref, m_ref, w_ref, x_ref):
    emb = lax.dot_general(
        basisT_ref[...], w_ref[...], (((0,), (0,)), ((), ())),
        preferred_element_type=jnp.float32, precision=_PREC)
    x_ref[...] = m_ref[...] * emb


@functools.lru_cache(maxsize=None)
def _edge_fuse(nE_chunk, base_blk, dR, dE, block_e):
    return pl.pallas_call(
        _edge_body,
        grid=(nE_chunk // block_e,),
        in_specs=[
            pl.BlockSpec((dR, block_e), lambda i: (0, i + base_blk)),
            pl.BlockSpec((block_e, dE), lambda i: (i + base_blk, 0)),
            pl.BlockSpec((dR, dE), lambda i: (0, 0)),
        ],
        out_specs=pl.BlockSpec((block_e, dE), lambda i: (i, 0)),
        out_shape=jax.ShapeDtypeStruct((nE_chunk, dE), jnp.float32),
    )


@functools.lru_cache(maxsize=None)
def _sc_scatter(nE_chunk, idx_off, nA, D):
    NC, NS = 2, 16
    NW = NC * NS
    epw = nE_chunk // NW
    block = 192
    nblk = epw // block
    tail = epw % block
    assert tail % 8 == 0 and tail <= 128
    rows_lo = (nA // NS) // 8 * 8
    rows_hi = nA - rows_lo * (NS - 1)

    mesh = plsc.VectorSubcoreMesh(core_axis_name="c", subcore_axis_name="s")

    scratch = [
        pltpu.VMEM((block, D), jnp.float32),
        pltpu.VMEM((block, D), jnp.float32),
        pltpu.VMEM((128,), jnp.int32),
        pltpu.VMEM((64,), jnp.int32),
        pltpu.VMEM((128,), jnp.int32),
        pltpu.VMEM((64,), jnp.int32),
        pltpu.SemaphoreType.DMA,
        pltpu.SemaphoreType.DMA,
        pltpu.VMEM_SHARED((nA, D), jnp.float32),
    ]
    if tail:
        scratch.insert(6, pltpu.VMEM((tail,), jnp.int32))

    @functools.partial(
        pl.kernel,
        mesh=mesh,
        out_type=jax.ShapeDtypeStruct((NC, nA, D), jnp.float32),
        scratch_types=scratch,
    )
    def sc_scatter(x_hbm, idx_hbm, out_hbm, *refs):
        if tail:
            xb0, xb1, ia0, ib0, ia1, ib1, it, sem0, sem1, acc = refs
        else:
            xb0, xb1, ia0, ib0, ia1, ib1, sem0, sem1, acc = refs
        xbufs = (xb0, xb1)
        ibufs = ((ia0, ib0), (ia1, ib1))
        sems = (sem0, sem1)
        c = lax.axis_index("c")
        s = lax.axis_index("s")
        wid = c * NS + s

        def _zrow(r, carry):
            for j in range(D // 16):
                xb0[r, pl.ds(j * 16, 16)] = jnp.zeros((16,), jnp.float32)
            return carry
        lax.fori_loop(0, block, _zrow, 0)

        r0 = s * rows_lo

        def _zero_rows(n):
            for k in range(n // block):
                pltpu.sync_copy(xb0, acc.at[pl.ds(r0 + k * block, block)])
            rem = n % block
            if rem:
                pltpu.sync_copy(xb0.at[pl.ds(0, rem)],
                                acc.at[pl.ds(r0 + (n // block) * block, rem)])

        @pl.when(s < NS - 1)
        def _():
            _zero_rows(rows_lo)

        @pl.when(s == NS - 1)
        def _():
            _zero_rows(rows_hi)

        plsc.subcore_barrier()

        base_w = wid * epw

        def _start(b):
            buf = b % 2
            ebase = base_w + b * block
            return (
                pltpu.async_copy(x_hbm.at[pl.ds(ebase, block)],
                                 xbufs[buf], sems[buf]),
                pltpu.async_copy(idx_hbm.at[pl.ds(idx_off + ebase, 128)],
                                 ibufs[buf][0], sems[buf]),
                pltpu.async_copy(idx_hbm.at[pl.ds(idx_off + ebase + 128, 64)],
                                 ibufs[buf][1], sems[buf]),
            )

        def _drain(handles):
            for hd in handles:
                hd.wait()

        def _scatter(b):
            buf = b % 2
            pltpu.sync_copy(xbufs[buf].at[pl.ds(0, 128)],
                            acc.at[ibufs[buf][0]], add=True)
            pltpu.sync_copy(xbufs[buf].at[pl.ds(128, 64)],
                            acc.at[ibufs[buf][1]], add=True)

        if nblk:
            pending = _start(0)
            for b in range(nblk):
                nxt = _start(b + 1) if b + 1 < nblk else None
                _drain(pending)
                _scatter(b)
                pending = nxt
        if tail:
            ebase = base_w + nblk * block
            pltpu.sync_copy(x_hbm.at[pl.ds(ebase, tail)],
                            xb0.at[pl.ds(0, tail)])
            pltpu.sync_copy(idx_hbm.at[pl.ds(idx_off + ebase, tail)], it)
            pltpu.sync_copy(xb0.at[pl.ds(0, tail)], acc.at[it], add=True)

        plsc.subcore_barrier()

        def _write_rows(n):
            nfull = n // block
            hs = [None, None]
            for k in range(nfull):
                off = r0 + k * block
                buf = k % 2
                if hs[buf] is not None:
                    hs[buf].wait()
                pltpu.sync_copy(acc.at[pl.ds(off, block)], xbufs[buf])
                hs[buf] = pltpu.async_copy(
                    xbufs[buf], out_hbm.at[c, pl.ds(off, block)], sems[buf])
            rem = n % block
            if rem:
                off = r0 + nfull * block
                buf = nfull % 2
                if hs[buf] is not None:
                    hs[buf].wait()
                pltpu.sync_copy(acc.at[pl.ds(off, rem)],
                                xbufs[buf].at[pl.ds(0, rem)])
                hs[buf] = pltpu.async_copy(
                    xbufs[buf].at[pl.ds(0, rem)],
                    out_hbm.at[c, pl.ds(off, rem)], sems[buf])
            for hd in hs:
                if hd is not None:
                    hd.wait()

        @pl.when(s < NS - 1)
        def _():
            _write_rows(rows_lo)

        @pl.when(s == NS - 1)
        def _():
            _write_rows(rows_hi)

    return sc_scatter


def _dot3(x, w):
    xh = x.astype(jnp.bfloat16)
    xl = (x - xh.astype(jnp.float32)).astype(jnp.bfloat16)
    wh = w.astype(jnp.bfloat16)
    wl = (w - wh.astype(jnp.float32)).astype(jnp.bfloat16)

    def d(a, b):
        return jax.lax.dot_general(
            a, b, (((1,), (0,)), ((), ())),
            preferred_element_type=jnp.float32)

    return d(xh, wh) + (d(xh, wl) + d(xl, wh))


def _res_layer(x, W1, W2):
    y = _dot3(x, W1)
    y = y * jax.nn.sigmoid(y)
    y = _dot3(y, W2)
    y = y * jax.nn.sigmoid(y)
    return (x + y) * _INV_SQRT2


def _mlp_body_fixed(n_parts):
    def body(*refs):
        part_refs = refs[:n_parts]
        h_ref, wpre_ref, wpost_ref, scale_ref, out_ref = refs[n_parts:]
        x = part_refs[0][0] + part_refs[0][1]
        for p in part_refs[1:]:
            x = x + p[0] + p[1]
        x = x * scale_ref[0, 0]
        for i in range(wpre_ref.shape[0]):
            x = _res_layer(x, wpre_ref[i, 0], wpre_ref[i, 1])
        x = (x + h_ref[...]) * _INV_SQRT2
        for i in range(wpost_ref.shape[0]):
            x = _res_layer(x, wpost_ref[i, 0], wpost_ref[i, 1])
        out_ref[...] = x
    return body


@functools.lru_cache(maxsize=None)
def _mlp(n_parts, nA, D, nPre, nPost, block_a):
    part_specs = [pl.BlockSpec((2, block_a, D), lambda i: (0, i, 0))
                  for _ in range(n_parts)]
    return pl.pallas_call(
        _mlp_body_fixed(n_parts),
        grid=(nA // block_a,),
        in_specs=part_specs + [
            pl.BlockSpec((block_a, D), lambda i: (i, 0)),
            pl.BlockSpec((nPre, 2, D, D), lambda i: (0, 0, 0, 0)),
            pl.BlockSpec((nPost, 2, D, D), lambda i: (0, 0, 0, 0)),
            pl.BlockSpec((1, 1), lambda i: (0, 0), memory_space=pltpu.SMEM),
        ],
        out_specs=pl.BlockSpec((block_a, D), lambda i: (i, 0)),
        out_shape=jax.ShapeDtypeStruct((nA, D), jnp.float32),
    )


_CHUNKS = (102400, 89600, 76800, 51200)
_BLOCK_E = 6400


def kernel(h, m, basis_rad, idx_atom, W_rbf, W_pre, W_post, scale):
    nA, D = h.shape
    nE, dR = basis_rad.shape
    idx32 = idx_atom.astype(jnp.int32)
    basisT = basis_rad.T
    partials = []
    base = 0
    for chunk in _CHUNKS:
        x_k = _edge_fuse(chunk, base // _BLOCK_E, dR, D, _BLOCK_E)(
            basisT, m, W_rbf)
        partials.append(_sc_scatter(chunk, base, nA, D)(x_k, idx32))
        base += chunk
    out = _mlp(len(partials), nA, D, W_pre.shape[0], W_post.shape[0], 5000)(
        *partials, h, W_pre, W_post, scale.reshape(1, 1))
    return out
